# baseline (speedup 1.0000x reference)
import functools

import jax
import jax.numpy as jnp
from jax.experimental import pallas as pl
from jax.experimental.pallas import tpu as pltpu

K_CODES = 1024
C_DIM = 256


def _vq_body(z_ref, e_ref, et_ref, zq_ref, loss_ref, *, precision):
    c, h, w = z_ref.shape[1:]
    z = z_ref[0].reshape(c, h * w)
    e = e_ref[...]

    e2 = jnp.sum(e * e, axis=1, keepdims=True)
    mm = jax.lax.dot_general(
        e, z, (((1,), (0,)), ((), ())),
        preferred_element_type=jnp.float32, precision=None)
    d = e2 - 2.0 * mm

    minval = jnp.min(d, axis=0, keepdims=True)
    rowi = jax.lax.broadcasted_iota(jnp.int32, d.shape, 0)
    idx = jnp.min(jnp.where(d == minval, rowi, K_CODES), axis=0)

    oh = (rowi == idx[None, :]).astype(jnp.float32)
    zq = jax.lax.dot_general(
        et_ref[...], oh, (((1,), (0,)), ((), ())),
        preferred_element_type=jnp.float32, precision=None)
    zq_ref[0] = zq.reshape(c, h, w)

    diff = zq - z
    partial = jnp.sum(diff * diff)
    @pl.when(pl.program_id(0) == 0)
    def _init():
        loss_ref[0, 0] = partial
    @pl.when(pl.program_id(0) != 0)
    def _acc():
        loss_ref[0, 0] += partial


@functools.partial(jax.jit, static_argnames=("precision",))
def _vq_call(z_e, e, et, precision="highest"):
    B, C, H, W = z_e.shape
    grid = (B,)
    body = functools.partial(_vq_body, precision=precision)
    zq, loss = pl.pallas_call(
        body,
        grid=grid,
        in_specs=[
            pl.BlockSpec((1, C, H, W), lambda b: (b, 0, 0, 0)),
            pl.BlockSpec((K_CODES, C), lambda b: (0, 0)),
            pl.BlockSpec((C, K_CODES), lambda b: (0, 0)),
        ],
        out_specs=[
            pl.BlockSpec((1, C, H, W), lambda b: (b, 0, 0, 0)),
            pl.BlockSpec(memory_space=pltpu.SMEM),
        ],
        out_shape=[
            jax.ShapeDtypeStruct((B, C, H, W), jnp.float32),
            jax.ShapeDtypeStruct((1, 1), jnp.float32),
        ],
    )(z_e, e, et)
    return zq, loss


def kernel(z_e, embedding):
    et = jnp.swapaxes(embedding, 0, 1)
    z_q_st, loss = _vq_call(z_e, embedding, et)
    beta = 0.25
    vq_loss = beta * loss[0, 0] / z_e.size
    return (z_q_st, vq_loss)


# speedup vs baseline: 1.3560x; 1.3560x over previous
import functools

import jax
import jax.numpy as jnp
from jax.experimental import pallas as pl
from jax.experimental.pallas import tpu as pltpu

K_CODES = 1024
C_DIM = 256


def _vq_body(z_ref, e_ref, et_ref, zq_ref, loss_ref, *, precision):
    c, h, w = z_ref.shape[1:]
    z = z_ref[0].reshape(c, h * w)
    e = e_ref[...]

    e2 = jnp.sum(e * e, axis=1, keepdims=True)
    mm = jax.lax.dot_general(
        e, z, (((1,), (0,)), ((), ())),
        preferred_element_type=jnp.float32, precision=None)
    d = e2 - 2.0 * mm

    minval = jnp.min(d, axis=0, keepdims=True)
    rowi = jax.lax.broadcasted_iota(jnp.int32, d.shape, 0)
    idx = jnp.min(jnp.where(d == minval, rowi, K_CODES), axis=0)

    oh = (rowi == idx[None, :]).astype(jnp.float32)
    zq = jax.lax.dot_general(
        et_ref[...], oh, (((1,), (0,)), ((), ())),
        preferred_element_type=jnp.float32, precision=None)
    zq_ref[0] = zq

    diff = zq - z
    partial = jnp.sum(diff * diff)
    @pl.when(pl.program_id(0) == 0)
    def _init():
        loss_ref[0, 0] = partial
    @pl.when(pl.program_id(0) != 0)
    def _acc():
        loss_ref[0, 0] += partial


@functools.partial(jax.jit, static_argnames=("precision",))
def _vq_call(z_e, e, et, precision="highest"):
    B, C, H, W = z_e.shape
    grid = (B,)
    body = functools.partial(_vq_body, precision=precision)
    zq, loss = pl.pallas_call(
        body,
        grid=grid,
        in_specs=[
            pl.BlockSpec((1, C, H, W), lambda b: (b, 0, 0, 0)),
            pl.BlockSpec((K_CODES, C), lambda b: (0, 0)),
            pl.BlockSpec((C, K_CODES), lambda b: (0, 0)),
        ],
        out_specs=[
            pl.BlockSpec((1, C, H * W), lambda b: (b, 0, 0)),
            pl.BlockSpec(memory_space=pltpu.SMEM),
        ],
        out_shape=[
            jax.ShapeDtypeStruct((B, C, H * W), jnp.float32),
            jax.ShapeDtypeStruct((1, 1), jnp.float32),
        ],
    )(z_e, e, et)
    return zq, loss


def kernel(z_e, embedding):
    B, C, H, W = z_e.shape
    et = jnp.swapaxes(embedding, 0, 1)
    zq_r, loss = _vq_call(z_e, embedding, et)
    z_q_st = zq_r.reshape(B, C, H, W)
    beta = 0.25
    vq_loss = beta * loss[0, 0] / z_e.size
    return (z_q_st, vq_loss)


# speedup vs baseline: 1.5085x; 1.1125x over previous
import functools

import jax
import jax.numpy as jnp
from jax.experimental import pallas as pl
from jax.experimental.pallas import tpu as pltpu

K_CODES = 1024
C_DIM = 256


def _vq_body(z_ref, e_ref, et_ref, zq_ref, loss_ref, *, precision):
    z = z_ref[0]
    e = e_ref[...]

    e2 = jnp.sum(e * e, axis=1, keepdims=True)
    mm = jax.lax.dot_general(
        e, z, (((1,), (0,)), ((), ())),
        preferred_element_type=jnp.float32, precision=None)
    d = e2 - 2.0 * mm

    minval = jnp.min(d, axis=0, keepdims=True)
    rowi = jax.lax.broadcasted_iota(jnp.int32, d.shape, 0)
    idx = jnp.min(jnp.where(d == minval, rowi, K_CODES), axis=0)

    oh = (rowi == idx[None, :]).astype(jnp.float32)
    zq = jax.lax.dot_general(
        et_ref[...], oh, (((1,), (0,)), ((), ())),
        preferred_element_type=jnp.float32, precision=None)
    zq_ref[0] = zq.reshape(zq_ref.shape[1:])

    diff = zq - z
    partial = jnp.sum(diff * diff)
    @pl.when(pl.program_id(0) == 0)
    def _init():
        loss_ref[0, 0] = partial
    @pl.when(pl.program_id(0) != 0)
    def _acc():
        loss_ref[0, 0] += partial


@functools.partial(jax.jit, static_argnames=("precision",))
def _vq_call(z_r, e, et, precision="highest"):
    B, C, P = z_r.shape
    H = W = 32
    grid = (B,)
    body = functools.partial(_vq_body, precision=precision)
    zq, loss = pl.pallas_call(
        body,
        grid=grid,
        in_specs=[
            pl.BlockSpec((1, C, P), lambda b: (b, 0, 0)),
            pl.BlockSpec((K_CODES, C), lambda b: (0, 0)),
            pl.BlockSpec((C, K_CODES), lambda b: (0, 0)),
        ],
        out_specs=[
            pl.BlockSpec((1, C, H, W), lambda b: (b, 0, 0, 0)),
            pl.BlockSpec(memory_space=pltpu.SMEM),
        ],
        out_shape=[
            jax.ShapeDtypeStruct((B, C, H, W), jnp.float32),
            jax.ShapeDtypeStruct((1, 1), jnp.float32),
        ],
    )(z_r, e, et)
    return zq, loss


def kernel(z_e, embedding):
    B, C, H, W = z_e.shape
    et = jnp.swapaxes(embedding, 0, 1)
    z_q_st, loss = _vq_call(z_e.reshape(B, C, H * W), embedding, et)
    beta = 0.25
    vq_loss = beta * loss[0, 0] / z_e.size
    return (z_q_st, vq_loss)


# speedup vs baseline: 2.3085x; 1.5303x over previous
import functools

import jax
import jax.numpy as jnp
from jax.experimental import pallas as pl
from jax.experimental.pallas import tpu as pltpu

K_CODES = 1024
C_DIM = 256


def _vq_body(z_ref, e_ref, et_ref, zq_ref, loss_ref):
    z = z_ref[0]
    e = e_ref[...]

    he2 = 0.5 * jnp.sum(e * e, axis=1, keepdims=True)
    mm = jax.lax.dot_general(
        e, z, (((1,), (0,)), ((), ())),
        preferred_element_type=jnp.float32, precision=None)
    t = mm - he2

    maxval = jnp.max(t, axis=0, keepdims=True)
    rowi = jax.lax.broadcasted_iota(jnp.int32, t.shape, 0)
    nrow = -rowi.astype(jnp.float32)
    nidx = jnp.max(jnp.where(t == maxval, nrow, -jnp.inf), axis=0)

    oh = (nrow == nidx[None, :]).astype(jnp.float32)
    zq = jax.lax.dot_general(
        et_ref[...], oh, (((1,), (0,)), ((), ())),
        preferred_element_type=jnp.float32, precision=None)
    zq_ref[0] = zq

    diff = zq - z
    partial = jnp.sum(diff * diff)
    @pl.when(pl.program_id(0) == 0)
    def _init():
        loss_ref[0, 0] = partial
    @pl.when(pl.program_id(0) != 0)
    def _acc():
        loss_ref[0, 0] += partial


@jax.jit
def _vq_call(z_r, e, et):
    B, C, P = z_r.shape
    zq_r, loss = pl.pallas_call(
        _vq_body,
        grid=(B,),
        in_specs=[
            pl.BlockSpec((1, C, P), lambda b: (b, 0, 0)),
            pl.BlockSpec((K_CODES, C), lambda b: (0, 0)),
            pl.BlockSpec((C, K_CODES), lambda b: (0, 0)),
        ],
        out_specs=[
            pl.BlockSpec((1, C, P), lambda b: (b, 0, 0)),
            pl.BlockSpec(memory_space=pltpu.SMEM),
        ],
        out_shape=[
            jax.ShapeDtypeStruct((B, C, P), jnp.float32),
            jax.ShapeDtypeStruct((1, 1), jnp.float32),
        ],
    )(z_r, e, et)
    return zq_r, loss


def kernel(z_e, embedding):
    B, C, H, W = z_e.shape
    z_r = z_e.reshape(B, C, H * W)
    et = jnp.swapaxes(embedding, 0, 1)
    zq_r, loss = _vq_call(z_r, embedding, et)
    z_q_st = zq_r.reshape(B, C, H, W)
    beta = 0.25
    vq_loss = beta * loss[0, 0] / z_e.size
    return (z_q_st, vq_loss)
